# baseline (speedup 1.0000x reference)
import jax
import jax.numpy as jnp
from jax.experimental import pallas as pl
from jax.experimental.pallas import tpu as pltpu

B = 16
IN_CH = 256
CH = 128
OUT_CH = 256
RES = 16
PIX = RES * RES
EMBED_DIM = 64
N_EMBED = 1024
N_PHYLO_CH = 64
N_LEVELS = 4
CB_PER_LEVEL = 8
FLAT_IN = N_PHYLO_CH * PIX
FLAT_CODE = EMBED_DIM * CB_PER_LEVEL * N_LEVELS
NBLK_IN = 16
NBLK_OUT = 32


def _silu(v):
    return v * jax.nn.sigmoid(v)


def _prologue_kernel(x_ref, ciw_ref, cib_ref, lng_ref, lnb_ref,
                     flat_ref, simg_ref):
    ciw = ciw_ref[...]
    cib = cib_ref[...]
    for b in range(B):
        sx = _silu(x_ref[b])
        h = jax.lax.dot_general(ciw, sx, (((1,), (0,)), ((), ())),
                                preferred_element_type=jnp.float32) + cib
        hp = h[:N_PHYLO_CH]
        hi = h[N_PHYLO_CH:]
        mu = jnp.mean(hp)
        var = jnp.mean((hp - mu) ** 2)
        hn = (hp - mu) * jax.lax.rsqrt(var + 1e-5)
        flat_ref[b] = hn * lng_ref[...] + lnb_ref[...]
        simg_ref[b] = _silu(hi)


def _mlp_in_kernel(flat_ref, w_ref, b_ref, z_ref):
    acc = jax.lax.dot_general(flat_ref[...], w_ref[...],
                              (((1,), (1,)), ((), ())),
                              preferred_element_type=jnp.float32)
    z_ref[...] = _silu(acc + b_ref[0])


def _quantize_kernel(zf_ref, cb_ref, zq_ref):
    zf = zf_ref[...]
    cb = cb_ref[...]
    ones = jnp.ones((1, EMBED_DIM), jnp.float32)
    cb_sq = jax.lax.dot_general(ones, cb * cb, (((1,), (1,)), ((), ())),
                                preferred_element_type=jnp.float32)
    cross = jax.lax.dot_general(zf, cb, (((1,), (1,)), ((), ())),
                                preferred_element_type=jnp.float32)
    d = cb_sq - 2.0 * cross
    dmin = jnp.min(d, axis=1, keepdims=True)
    iota = jax.lax.broadcasted_iota(jnp.int32, (B * 32, N_EMBED), 1)
    idx = jnp.min(jnp.where(d <= dmin, iota, N_EMBED), axis=1, keepdims=True)
    oh = (iota == idx).astype(jnp.float32)
    zq_ref[...] = jax.lax.dot_general(oh, cb, (((1,), (0,)), ((), ())),
                                      preferred_element_type=jnp.float32)


def _mlp_out_kernel(zq_ref, w_ref, b_ref, h_ref):
    acc = jax.lax.dot_general(zq_ref[...], w_ref[...],
                              (((1,), (1,)), ((), ())),
                              preferred_element_type=jnp.float32)
    h_ref[...] = _silu(acc + b_ref[0])


def _epilogue_kernel(hout_ref, simg_ref, cow_ref, cob_ref, out_ref):
    w_p = cow_ref[:, :N_PHYLO_CH]
    w_i = cow_ref[:, N_PHYLO_CH:]
    cob = cob_ref[...]
    for b in range(B):
        sp = _silu(hout_ref[b])
        o = (jax.lax.dot_general(w_p, sp, (((1,), (0,)), ((), ())),
                                 preferred_element_type=jnp.float32)
             + jax.lax.dot_general(w_i, simg_ref[b], (((1,), (0,)), ((), ())),
                                   preferred_element_type=jnp.float32)
             + cob)
        out_ref[b] = o


def kernel(x, conv_in_w, conv_in_b, ln_g, ln_b, mlp_in_w, mlp_in_b,
           codebook, mlp_out_w, mlp_out_b, conv_out_w, conv_out_b):
    f32 = jnp.float32
    x_r = x.reshape(B, IN_CH, PIX)
    cib = conv_in_b.reshape(CH, 1)
    lng = ln_g.reshape(N_PHYLO_CH, PIX)
    lnb = ln_b.reshape(N_PHYLO_CH, PIX)

    flat3, simg = pl.pallas_call(
        _prologue_kernel,
        out_shape=(jax.ShapeDtypeStruct((B, N_PHYLO_CH, PIX), f32),
                   jax.ShapeDtypeStruct((B, CH - N_PHYLO_CH, PIX), f32)),
    )(x_r, conv_in_w, cib, lng, lnb)

    flat = flat3.reshape(B, FLAT_IN)
    b_in = mlp_in_b.reshape(NBLK_IN, 1, FLAT_CODE // NBLK_IN)
    z = pl.pallas_call(
        _mlp_in_kernel,
        grid=(NBLK_IN,),
        in_specs=[
            pl.BlockSpec((B, FLAT_IN), lambda i: (0, 0)),
            pl.BlockSpec((FLAT_CODE // NBLK_IN, FLAT_IN), lambda i: (i, 0)),
            pl.BlockSpec((1, 1, FLAT_CODE // NBLK_IN), lambda i: (i, 0, 0)),
        ],
        out_specs=pl.BlockSpec((B, FLAT_CODE // NBLK_IN), lambda i: (0, i)),
        out_shape=jax.ShapeDtypeStruct((B, FLAT_CODE), f32),
        compiler_params=pltpu.CompilerParams(
            dimension_semantics=("arbitrary",)),
    )(flat, mlp_in_w, b_in)

    zf = z.reshape(B, EMBED_DIM, 32).transpose(0, 2, 1).reshape(B * 32, EMBED_DIM)
    zq_rows = pl.pallas_call(
        _quantize_kernel,
        out_shape=jax.ShapeDtypeStruct((B * 32, EMBED_DIM), f32),
    )(zf, codebook)
    zq = zq_rows.reshape(B, 32, EMBED_DIM).transpose(0, 2, 1).reshape(B, FLAT_CODE)

    b_out = mlp_out_b.reshape(NBLK_OUT, 1, FLAT_IN // NBLK_OUT)
    hout = pl.pallas_call(
        _mlp_out_kernel,
        grid=(NBLK_OUT,),
        in_specs=[
            pl.BlockSpec((B, FLAT_CODE), lambda i: (0, 0)),
            pl.BlockSpec((FLAT_IN // NBLK_OUT, FLAT_CODE), lambda i: (i, 0)),
            pl.BlockSpec((1, 1, FLAT_IN // NBLK_OUT), lambda i: (i, 0, 0)),
        ],
        out_specs=pl.BlockSpec((B, FLAT_IN // NBLK_OUT), lambda i: (0, i)),
        out_shape=jax.ShapeDtypeStruct((B, FLAT_IN), f32),
        compiler_params=pltpu.CompilerParams(
            dimension_semantics=("arbitrary",)),
    )(zq, mlp_out_w, b_out)

    cob = conv_out_b.reshape(OUT_CH, 1)
    out = pl.pallas_call(
        _epilogue_kernel,
        out_shape=jax.ShapeDtypeStruct((B, OUT_CH, PIX), f32),
    )(hout.reshape(B, N_PHYLO_CH, PIX), simg, conv_out_w, cob)
    return out.reshape(B, OUT_CH, RES, RES)


# speedup vs baseline: 1.1675x; 1.1675x over previous
import jax
import jax.numpy as jnp
from jax.experimental import pallas as pl
from jax.experimental.pallas import tpu as pltpu

B = 16
IN_CH = 256
CH = 128
OUT_CH = 256
RES = 16
PIX = RES * RES
EMBED_DIM = 64
N_EMBED = 1024
N_PHYLO_CH = 64
FLAT_IN = N_PHYLO_CH * PIX
FLAT_CODE = 2048
NBI = 16
NBO = 16
BS_IN = FLAT_CODE // NBI
BS_OUT = FLAT_IN // NBO
STEPS = 1 + NBI + 1 + NBO
QSTEP = NBI + 1


def _silu(v):
    return v * jax.nn.sigmoid(v)


def _fused_kernel(x_ref, ciw_ref, cib_ref, lng_ref, lnb_ref,
                  wi_ref, bi_ref, cb_ref, wo_ref, bo_ref,
                  hout_ref, simg_ref,
                  flat_s, z_s, zq_s):
    i = pl.program_id(0)

    @pl.when(i == 0)
    def _prologue():
        ciw = ciw_ref[...]
        cib = cib_ref[...]
        for b in range(B):
            sx = _silu(x_ref[b])
            h = jax.lax.dot_general(ciw, sx, (((1,), (0,)), ((), ())),
                                    preferred_element_type=jnp.float32) + cib
            hp = h[:N_PHYLO_CH]
            mu = jnp.mean(hp)
            var = jnp.mean((hp - mu) ** 2)
            hn = (hp - mu) * jax.lax.rsqrt(var + 1e-5)
            flat_s[b] = hn * lng_ref[...] + lnb_ref[...]
            simg_ref[b] = _silu(h[N_PHYLO_CH:])

    @pl.when((i >= 1) & (i <= NBI))
    def _mlp_in():
        acc = jax.lax.dot_general(
            flat_s[...].reshape(B, FLAT_IN), wi_ref[...],
            (((1,), (1,)), ((), ())), preferred_element_type=jnp.float32)
        z_s[:, pl.ds((i - 1) * BS_IN, BS_IN)] = _silu(acc + bi_ref[0])

    @pl.when(i == QSTEP)
    def _quantize():
        z = z_s[...]
        zf = jnp.transpose(z.reshape(B, EMBED_DIM, 32),
                           (0, 2, 1)).reshape(B * 32, EMBED_DIM)
        cb = cb_ref[...]
        ones = jnp.ones((1, EMBED_DIM), jnp.float32)
        cb_sq = jax.lax.dot_general(ones, cb * cb, (((1,), (1,)), ((), ())),
                                    preferred_element_type=jnp.float32)
        cross = jax.lax.dot_general(zf, cb, (((1,), (1,)), ((), ())),
                                    preferred_element_type=jnp.float32)
        d = cb_sq - 2.0 * cross
        dmin = jnp.min(d, axis=1, keepdims=True)
        iota = jax.lax.broadcasted_iota(jnp.int32, (B * 32, N_EMBED), 1)
        idx = jnp.min(jnp.where(d <= dmin, iota, N_EMBED), axis=1,
                      keepdims=True)
        oh = (iota == idx).astype(jnp.float32)
        zq_rows = jax.lax.dot_general(oh, cb, (((1,), (0,)), ((), ())),
                                      preferred_element_type=jnp.float32)
        zq_s[...] = jnp.transpose(zq_rows.reshape(B, 32, EMBED_DIM),
                                  (0, 2, 1)).reshape(B, FLAT_CODE)

    @pl.when(i >= QSTEP + 1)
    def _mlp_out():
        acc = jax.lax.dot_general(zq_s[...], wo_ref[...],
                                  (((1,), (1,)), ((), ())),
                                  preferred_element_type=jnp.float32)
        hout_ref[:, pl.ds((i - QSTEP - 1) * BS_OUT, BS_OUT)] = \
            _silu(acc + bo_ref[0])


def _epilogue_kernel(hout_ref, simg_ref, cow_ref, cob_ref, out_ref):
    w_p = cow_ref[:, :N_PHYLO_CH]
    w_i = cow_ref[:, N_PHYLO_CH:]
    cob = cob_ref[...]
    for b in range(B):
        sp = _silu(hout_ref[b])
        out_ref[b] = (
            jax.lax.dot_general(w_p, sp, (((1,), (0,)), ((), ())),
                                preferred_element_type=jnp.float32)
            + jax.lax.dot_general(w_i, simg_ref[b], (((1,), (0,)), ((), ())),
                                  preferred_element_type=jnp.float32)
            + cob)


def kernel(x, conv_in_w, conv_in_b, ln_g, ln_b, mlp_in_w, mlp_in_b,
           codebook, mlp_out_w, mlp_out_b, conv_out_w, conv_out_b):
    f32 = jnp.float32
    x_r = x.reshape(B, IN_CH, PIX)
    cib = conv_in_b.reshape(CH, 1)
    lng = ln_g.reshape(N_PHYLO_CH, PIX)
    lnb = ln_b.reshape(N_PHYLO_CH, PIX)
    b_in = mlp_in_b.reshape(NBI, 1, BS_IN)
    b_out = mlp_out_b.reshape(NBO, 1, BS_OUT)

    hout, simg = pl.pallas_call(
        _fused_kernel,
        grid=(STEPS,),
        in_specs=[
            pl.BlockSpec((B, IN_CH, PIX), lambda i: (0, 0, 0)),
            pl.BlockSpec((CH, IN_CH), lambda i: (0, 0)),
            pl.BlockSpec((CH, 1), lambda i: (0, 0)),
            pl.BlockSpec((N_PHYLO_CH, PIX), lambda i: (0, 0)),
            pl.BlockSpec((N_PHYLO_CH, PIX), lambda i: (0, 0)),
            pl.BlockSpec((BS_IN, FLAT_IN),
                         lambda i: (jnp.clip(i - 1, 0, NBI - 1), 0)),
            pl.BlockSpec((1, 1, BS_IN),
                         lambda i: (jnp.clip(i - 1, 0, NBI - 1), 0, 0)),
            pl.BlockSpec((N_EMBED, EMBED_DIM), lambda i: (0, 0)),
            pl.BlockSpec((BS_OUT, FLAT_CODE),
                         lambda i: (jnp.clip(i - QSTEP - 1, 0, NBO - 1), 0)),
            pl.BlockSpec((1, 1, BS_OUT),
                         lambda i: (jnp.clip(i - QSTEP - 1, 0, NBO - 1), 0, 0)),
        ],
        out_specs=(pl.BlockSpec((B, FLAT_IN), lambda i: (0, 0)),
                   pl.BlockSpec((B, CH - N_PHYLO_CH, PIX), lambda i: (0, 0, 0))),
        out_shape=(jax.ShapeDtypeStruct((B, FLAT_IN), f32),
                   jax.ShapeDtypeStruct((B, CH - N_PHYLO_CH, PIX), f32)),
        scratch_shapes=[
            pltpu.VMEM((B, N_PHYLO_CH, PIX), f32),
            pltpu.VMEM((B, FLAT_CODE), f32),
            pltpu.VMEM((B, FLAT_CODE), f32),
        ],
        compiler_params=pltpu.CompilerParams(
            dimension_semantics=("arbitrary",)),
    )(x_r, conv_in_w, cib, lng, lnb, mlp_in_w, b_in, codebook,
      mlp_out_w, b_out)

    cob = conv_out_b.reshape(OUT_CH, 1)
    out = pl.pallas_call(
        _epilogue_kernel,
        out_shape=jax.ShapeDtypeStruct((B, OUT_CH, PIX), f32),
    )(hout.reshape(B, N_PHYLO_CH, PIX), simg, conv_out_w, cob)
    return out.reshape(B, OUT_CH, RES, RES)
